# Initial kernel scaffold; baseline (speedup 1.0000x reference)
import jax
import jax.numpy as jnp
from jax.experimental import pallas as pl


def kernel(x, W1, b1, W2, b2, W3, b3):
    raise NotImplementedError("write your pallas kernel here")


# speedup vs baseline: 9.7746x; 9.7746x over previous
import functools

import jax
import jax.numpy as jnp
import numpy as np
from jax.experimental import pallas as pl
from jax.experimental.pallas import tpu as pltpu

KK = 9
NN = 64
SC = 2
HH = 224
LL = 112 * 112
TL = 896
NBLK = LL // TL

_SIDX = tuple(
    int(v) for v in np.round(
        np.arange(NN, dtype=np.float32)
        * (np.float32(LL - 1) / np.float32(NN - 1))).astype(np.int32))


def _unshuffle(x, s):
    B, C, H, W = x.shape
    x = x.reshape(B, C, H // s, s, W // s, s)
    x = x.transpose(0, 1, 3, 5, 2, 4)
    return x.reshape(B, C * s * s, H // s, W // s)


def _shuffle(x, s):
    B, C, H, W = x.shape
    x = x.reshape(B, C // (s * s), s, s, H, W)
    x = x.transpose(0, 1, 4, 2, 5, 3)
    return x.reshape(B, C // (s * s), H * s, W * s)


def _layer_kernel(relu, tok_ref, samp_ref, wt_ref, bias_ref, out_ref):
    tok = tok_ref[0]
    samp = samp_ref[0]
    cross = jax.lax.dot_general(
        tok, samp, (((1,), (1,)), ((), ())),
        preferred_element_type=jnp.float32)
    t2 = jnp.sum(tok * tok, axis=1, keepdims=True)
    s2 = jnp.sum(samp * samp, axis=1)
    d2 = (t2 - 2.0 * cross) + s2[None, :]
    iota = jax.lax.broadcasted_iota(jnp.int32, (TL, NN), 1)
    acc = jnp.zeros((TL, wt_ref.shape[2]), jnp.float32)
    for k in range(KK):
        m = jnp.min(d2, axis=1, keepdims=True)
        ismin = d2 <= m
        first = jnp.min(jnp.where(ismin, iota, NN), axis=1, keepdims=True)
        oh = (iota == first).astype(jnp.float32)
        d2 = jnp.where(iota == first, jnp.float32(jnp.inf), d2)
        g = jax.lax.dot_general(
            oh, samp, (((1,), (0,)), ((), ())),
            precision=jax.lax.Precision.HIGHEST,
            preferred_element_type=jnp.float32)
        acc = acc + jax.lax.dot_general(
            g, wt_ref[k], (((1,), (0,)), ((), ())),
            preferred_element_type=jnp.float32)
    acc = acc + bias_ref[0][None, :]
    if relu:
        acc = jnp.maximum(acc, 0.0)
    out_ref[0] = acc


def _nn_layer(tokens, Wk, b, relu):
    B, L, Cin = tokens.shape
    Cout = Wk.shape[0]
    samp = tokens[:, _SIDX, :]
    Wt = jnp.transpose(Wk, (2, 1, 0))
    out = pl.pallas_call(
        functools.partial(_layer_kernel, relu),
        grid=(B, NBLK),
        in_specs=[
            pl.BlockSpec((1, TL, Cin), lambda b_, i: (b_, i, 0)),
            pl.BlockSpec((1, NN, Cin), lambda b_, i: (b_, 0, 0)),
            pl.BlockSpec((KK, Cin, Cout), lambda b_, i: (0, 0, 0)),
            pl.BlockSpec((1, Cout), lambda b_, i: (0, 0)),
        ],
        out_specs=pl.BlockSpec((1, TL, Cout), lambda b_, i: (b_, i, 0)),
        out_shape=jax.ShapeDtypeStruct((B, L, Cout), jnp.float32),
    )(tokens, samp, Wt, b.reshape(1, Cout))
    return out


def kernel(x, W1, b1, W2, b2, W3, b3):
    B = x.shape[0]
    H = x.shape[2]
    W = x.shape[3]
    ys = jnp.linspace(-1.0, 1.0, H)
    xs = jnp.linspace(-1.0, 1.0, W)
    gy, gx = jnp.meshgrid(ys, xs, indexing='ij')
    loc = jnp.stack([gy, gx])[None]
    locu = _unshuffle(loc, SC)
    L = (H // SC) * (W // SC)
    loct = jnp.broadcast_to(
        locu.reshape(1, 2 * SC * SC, L).transpose(0, 2, 1), (B, L, 2 * SC * SC))

    xt = _unshuffle(x, SC)
    t = jnp.concatenate(
        [xt.reshape(B, -1, L).transpose(0, 2, 1), loct], axis=-1)

    o1 = _nn_layer(t, W1, b1, relu=True)
    t = jnp.concatenate([o1, loct], axis=-1)
    o2 = _nn_layer(t, W2, b2, relu=True)
    t = jnp.concatenate([o2, loct], axis=-1)
    o3 = _nn_layer(t, W3, b3, relu=False)

    Cout = o3.shape[-1]
    out = o3.transpose(0, 2, 1).reshape(B, Cout, H // SC, W // SC)
    return _shuffle(out, SC)
